# baseline (speedup 1.0000x reference)
import jax
import jax.numpy as jnp
from jax.experimental import pallas as pl
from jax.experimental.pallas import tpu as pltpu

_BM = 200
_NBUF = 4


def _gconv_stream_kernel(flt_hbm, x_ref, w_ref, o_ref, buf, sems):
    n_rows = flt_hbm.shape[0]
    nchunks = n_rows // _BM

    def copy(c, slot):
        return pltpu.make_async_copy(
            flt_hbm.at[pl.ds(c * _BM, _BM), :], buf.at[slot], sems.at[slot])

    for s in range(min(_NBUF - 1, nchunks)):
        copy(s, s).start()

    def body(c, _):
        slot = jax.lax.rem(c, _NBUF)
        nxt = c + _NBUF - 1

        @pl.when(nxt < nchunks)
        def _():
            copy(nxt, jax.lax.rem(nxt, _NBUF)).start()

        copy(c, slot).wait()
        acc = jnp.dot(buf[slot], x_ref[...],
                      preferred_element_type=jnp.float32)
        lin = jax.lax.dot_general(
            acc, w_ref[...], (((1,), (1,)), ((), ())),
            preferred_element_type=jnp.float32)
        o_ref[pl.ds(c * _BM, _BM), :] = jnp.tanh(lin)
        return 0

    jax.lax.fori_loop(0, nchunks, body, 0)


def kernel(inputs, flt, W):
    n_rows, n_cols = flt.shape
    d_in = inputs.shape[1]
    d_out = W.shape[0]
    return pl.pallas_call(
        _gconv_stream_kernel,
        in_specs=[
            pl.BlockSpec(memory_space=pl.ANY),
            pl.BlockSpec((n_cols, d_in), lambda: (0, 0)),
            pl.BlockSpec((d_out, d_in), lambda: (0, 0)),
        ],
        out_specs=pl.BlockSpec((n_rows, d_out), lambda: (0, 0)),
        out_shape=jax.ShapeDtypeStruct((n_rows, d_out), jnp.float32),
        scratch_shapes=[
            pltpu.VMEM((_NBUF, _BM, n_cols), jnp.float32),
            pltpu.SemaphoreType.DMA((_NBUF,)),
        ],
    )(flt, inputs, W)
